# baseline (speedup 1.0000x reference)
import functools

import jax
import jax.numpy as jnp
from jax import lax
from jax.experimental import pallas as pl
from jax.experimental.pallas import tpu as pltpu
from jax.experimental.pallas import tpu_sc as plsc

N = 10000
E = 320000
D = 128

NS = 16
CHUNK = 128
PAD_CHUNKS = 2560
TILE_CHUNKS = PAD_CHUNKS // NS
DEG_W = 32

BLK = 80
NBLK = N // BLK
BLK_PER_TILE = -(-NBLK // NS)

NBUF = 4
NBODY = TILE_CHUNKS // NBUF


def _sc_body(x_hbm, src0, dst0, src1, dst1,
             acc0_hbm, deg0_hbm, acc1_hbm, deg1_hbm,
             src_v, dst_v, rows_v, ones_v, acc_s, deg_s,
             gs0, gs1, gs2, gs3, ss0, ss1, ss2, ss3):
    gsem = [gs0, gs1, gs2, gs3]
    ssem = [ss0, ss1, ss2, ss3]
    c = lax.axis_index("c")
    s = lax.axis_index("s")

    @pl.loop(0, CHUNK)
    def _fill(i):
        ones_v[i, :] = jnp.ones((DEG_W,), jnp.bfloat16)

    @pl.loop(0, BLK_PER_TILE)
    def _init(k):
        bid = s * BLK_PER_TILE + k

        @pl.when(bid < NBLK)
        def _():
            off = pl.multiple_of(bid * BLK, 8)
            pltpu.sync_copy(x_hbm.at[pl.ds(off, BLK)],
                            acc_s.at[pl.ds(off, BLK)])
            pltpu.sync_copy(ones_v.at[pl.ds(0, BLK)],
                            deg_s.at[pl.ds(off, BLK)])

    span = pl.multiple_of(s * TILE_CHUNKS, 8)

    @pl.when(c == 0)
    def _stage0():
        pltpu.sync_copy(src0.at[pl.ds(span, TILE_CHUNKS)], src_v)
        pltpu.sync_copy(dst0.at[pl.ds(span, TILE_CHUNKS)], dst_v)

    @pl.when(c == 1)
    def _stage1():
        pltpu.sync_copy(src1.at[pl.ds(span, TILE_CHUNKS)], src_v)
        pltpu.sync_copy(dst1.at[pl.ds(span, TILE_CHUNKS)], dst_v)

    plsc.subcore_barrier()

    for b in range(NBUF):
        pltpu.async_copy(x_hbm.at[src_v.at[b]], rows_v.at[b], gsem[b])

    @pl.loop(0, NBODY)
    def _body(t):
        base = t * NBUF
        for b in range(NBUF):
            k = base + b
            pltpu.make_async_copy(
                x_hbm.at[src_v.at[k]], rows_v.at[b], gsem[b]).wait()
            pltpu.async_copy(rows_v.at[b], acc_s.at[dst_v.at[k]],
                             ssem[b], add=True)
            pltpu.async_copy(ones_v, deg_s.at[dst_v.at[k]],
                             ssem[b], add=True)
        for b in range(NBUF):
            k = base + b
            pltpu.make_async_copy(
                rows_v.at[b], acc_s.at[dst_v.at[k]], ssem[b]).wait()
            pltpu.make_async_copy(
                ones_v, deg_s.at[dst_v.at[k]], ssem[b]).wait()

            @pl.when(t < NBODY - 1)
            def _():
                kn = base + NBUF + b
                pltpu.async_copy(x_hbm.at[src_v.at[kn]],
                                 rows_v.at[b], gsem[b])

    plsc.subcore_barrier()

    def copy_out(acc_hbm, deg_hbm):
        @pl.loop(0, BLK_PER_TILE)
        def _out(k):
            bid = s * BLK_PER_TILE + k

            @pl.when(bid < NBLK)
            def _():
                off = pl.multiple_of(bid * BLK, 8)
                pltpu.sync_copy(acc_s.at[pl.ds(off, BLK)],
                                acc_hbm.at[pl.ds(off, BLK)])
                pltpu.sync_copy(deg_s.at[pl.ds(off, BLK)],
                                deg_hbm.at[pl.ds(off, BLK)])

    @pl.when(c == 0)
    def _out0():
        copy_out(acc0_hbm, deg0_hbm)

    @pl.when(c == 1)
    def _out1():
        copy_out(acc1_hbm, deg1_hbm)


_sc_aggregate = functools.partial(
    pl.kernel,
    out_type=(
        jax.ShapeDtypeStruct((N, D), jnp.bfloat16),
        jax.ShapeDtypeStruct((N, DEG_W), jnp.bfloat16),
        jax.ShapeDtypeStruct((N, D), jnp.bfloat16),
        jax.ShapeDtypeStruct((N, DEG_W), jnp.bfloat16),
    ),
    mesh=plsc.VectorSubcoreMesh(core_axis_name="c", subcore_axis_name="s"),
    compiler_params=pltpu.CompilerParams(use_tc_tiling_on_sc=False),
    scratch_types=[
        pltpu.VMEM((TILE_CHUNKS, CHUNK), jnp.int32),
        pltpu.VMEM((TILE_CHUNKS, CHUNK), jnp.int32),
        pltpu.VMEM((NBUF, CHUNK, D), jnp.bfloat16),
        pltpu.VMEM((CHUNK, DEG_W), jnp.bfloat16),
        pltpu.VMEM_SHARED((N + 8, D), jnp.bfloat16),
        pltpu.VMEM_SHARED((N + 8, DEG_W), jnp.bfloat16),
        pltpu.SemaphoreType.DMA,
        pltpu.SemaphoreType.DMA,
        pltpu.SemaphoreType.DMA,
        pltpu.SemaphoreType.DMA,
        pltpu.SemaphoreType.DMA,
        pltpu.SemaphoreType.DMA,
        pltpu.SemaphoreType.DMA,
        pltpu.SemaphoreType.DMA,
    ],
)(_sc_body)


ROW_BLK = 400


def _tc_body(acc0, deg0, acc1, deg1, w0, w1, b0, b1, out):
    h0 = acc0[:, :].astype(jnp.float32) / deg0[:, 0:1].astype(jnp.float32)
    h1 = acc1[:, :].astype(jnp.float32) / deg1[:, 0:1].astype(jnp.float32)
    y = (jnp.dot(h0, w0[:, :], preferred_element_type=jnp.float32)
         + jnp.dot(h1, w1[:, :], preferred_element_type=jnp.float32))
    out[:, :] = 0.5 * y + 0.5 * (b0[:, :] + b1[:, :])


def _tc_combine(acc0, deg0, acc1, deg1, w0, w1, b0, b1):
    grid = (N // ROW_BLK,)
    row_spec = pl.BlockSpec((ROW_BLK, D), lambda i: (i, 0))
    deg_spec = pl.BlockSpec((ROW_BLK, DEG_W), lambda i: (i, 0))
    full_spec = pl.BlockSpec((D, D), lambda i: (0, 0))
    bias_spec = pl.BlockSpec((1, D), lambda i: (0, 0))
    return pl.pallas_call(
        _tc_body,
        grid=grid,
        in_specs=[row_spec, deg_spec, row_spec, deg_spec,
                  full_spec, full_spec, bias_spec, bias_spec],
        out_specs=row_spec,
        out_shape=jax.ShapeDtypeStruct((N, D), jnp.float32),
    )(acc0, deg0, acc1, deg1, w0, w1, b0, b1)


def _pad_edges(edge_index):
    pad = PAD_CHUNKS * CHUNK - E
    src = jnp.concatenate(
        [edge_index[0], jnp.zeros((pad,), jnp.int32)]).reshape(PAD_CHUNKS, CHUNK)
    dst = jnp.concatenate(
        [edge_index[1], jnp.full((pad,), N, jnp.int32)]).reshape(PAD_CHUNKS, CHUNK)
    return src, dst


def kernel(x, edge_index_view0, edge_index_view1,
           W_view0, b_view0, W_view1, b_view1):
    src0, dst0 = _pad_edges(edge_index_view0)
    src1, dst1 = _pad_edges(edge_index_view1)
    x_bf = x.astype(jnp.bfloat16)
    acc0, deg0, acc1, deg1 = _sc_aggregate(x_bf, src0, dst0, src1, dst1)
    return _tc_combine(acc0, deg0, acc1, deg1,
                       W_view0, W_view1,
                       b_view0.reshape(1, D), b_view1.reshape(1, D))


# speedup vs baseline: 1.0886x; 1.0886x over previous
import functools

import jax
import jax.numpy as jnp
from jax import lax
from jax.experimental import pallas as pl
from jax.experimental.pallas import tpu as pltpu
from jax.experimental.pallas import tpu_sc as plsc

N = 10000
E = 320000
D = 128
HALF = 64

NS = 16
CHUNK = 128
PAD_CHUNKS = 2560
TILE_CHUNKS = PAD_CHUNKS // NS
DEG_W = 32

BLK = 80
NBLK = N // BLK
BLK_PER_TILE = -(-NBLK // NS)

NBUF = 4
NBODY = TILE_CHUNKS // NBUF


def _sc_body(x_hbm, src0, dst0, src1, dst1,
             acc0_hbm, deg0_hbm, acc1_hbm, deg1_hbm,
             src_v, dst_v, rows_v, ones_v, x_s, acc_s, deg_s,
             gs0, gs1, gs2, gs3, ss0, ss1, ss2, ss3):
    gsem = [gs0, gs1, gs2, gs3]
    ssem = [ss0, ss1, ss2, ss3]
    c = lax.axis_index("c")
    s = lax.axis_index("s")

    @pl.loop(0, CHUNK)
    def _fill(i):
        ones_v[i, :] = jnp.ones((DEG_W,), jnp.bfloat16)

    span = pl.multiple_of(s * TILE_CHUNKS, 8)

    @pl.when(c == 0)
    def _stage0():
        pltpu.sync_copy(src0.at[pl.ds(span, TILE_CHUNKS)], src_v)
        pltpu.sync_copy(dst0.at[pl.ds(span, TILE_CHUNKS)], dst_v)

    @pl.when(c == 1)
    def _stage1():
        pltpu.sync_copy(src1.at[pl.ds(span, TILE_CHUNKS)], src_v)
        pltpu.sync_copy(dst1.at[pl.ds(span, TILE_CHUNKS)], dst_v)

    for h in range(2):
        col = h * HALF

        @pl.loop(0, BLK_PER_TILE)
        def _init(k):
            bid = s * BLK_PER_TILE + k

            @pl.when(bid < NBLK)
            def _():
                off = pl.multiple_of(bid * BLK, 8)
                pltpu.sync_copy(
                    x_hbm.at[pl.ds(off, BLK), pl.ds(col, HALF)],
                    x_s.at[pl.ds(off, BLK)])
                pltpu.sync_copy(x_s.at[pl.ds(off, BLK)],
                                acc_s.at[pl.ds(off, BLK)])
                if h == 0:
                    pltpu.sync_copy(ones_v.at[pl.ds(0, BLK)],
                                    deg_s.at[pl.ds(off, BLK)])

        plsc.subcore_barrier()

        for b in range(NBUF):
            pltpu.async_copy(x_s.at[src_v.at[b]], rows_v.at[b], gsem[b])

        @pl.loop(0, NBODY)
        def _body(t):
            base = t * NBUF
            for b in range(NBUF):
                k = base + b
                pltpu.make_async_copy(
                    x_s.at[src_v.at[k]], rows_v.at[b], gsem[b]).wait()
                pltpu.async_copy(rows_v.at[b], acc_s.at[dst_v.at[k]],
                                 ssem[b], add=True)
                if h == 0:
                    pltpu.async_copy(ones_v, deg_s.at[dst_v.at[k]],
                                     ssem[b], add=True)
            for b in range(NBUF):
                k = base + b
                pltpu.make_async_copy(
                    rows_v.at[b], acc_s.at[dst_v.at[k]], ssem[b]).wait()
                if h == 0:
                    pltpu.make_async_copy(
                        ones_v, deg_s.at[dst_v.at[k]], ssem[b]).wait()

                @pl.when(t < NBODY - 1)
                def _():
                    kn = base + NBUF + b
                    pltpu.async_copy(x_s.at[src_v.at[kn]],
                                     rows_v.at[b], gsem[b])

        plsc.subcore_barrier()

        def copy_out(acc_hbm, deg_hbm):
            @pl.loop(0, BLK_PER_TILE)
            def _out(k):
                bid = s * BLK_PER_TILE + k

                @pl.when(bid < NBLK)
                def _():
                    off = pl.multiple_of(bid * BLK, 8)
                    pltpu.sync_copy(
                        acc_s.at[pl.ds(off, BLK)],
                        acc_hbm.at[pl.ds(off, BLK), pl.ds(col, HALF)])
                    if h == 0:
                        pltpu.sync_copy(deg_s.at[pl.ds(off, BLK)],
                                        deg_hbm.at[pl.ds(off, BLK)])

        @pl.when(c == 0)
        def _out0():
            copy_out(acc0_hbm, deg0_hbm)

        @pl.when(c == 1)
        def _out1():
            copy_out(acc1_hbm, deg1_hbm)


_sc_aggregate = functools.partial(
    pl.kernel,
    out_type=(
        jax.ShapeDtypeStruct((N, D), jnp.bfloat16),
        jax.ShapeDtypeStruct((N, DEG_W), jnp.bfloat16),
        jax.ShapeDtypeStruct((N, D), jnp.bfloat16),
        jax.ShapeDtypeStruct((N, DEG_W), jnp.bfloat16),
    ),
    mesh=plsc.VectorSubcoreMesh(core_axis_name="c", subcore_axis_name="s"),
    compiler_params=pltpu.CompilerParams(use_tc_tiling_on_sc=False),
    scratch_types=[
        pltpu.VMEM((TILE_CHUNKS, CHUNK), jnp.int32),
        pltpu.VMEM((TILE_CHUNKS, CHUNK), jnp.int32),
        pltpu.VMEM((NBUF, CHUNK, HALF), jnp.bfloat16),
        pltpu.VMEM((CHUNK, DEG_W), jnp.bfloat16),
        pltpu.VMEM_SHARED((N, HALF), jnp.bfloat16),
        pltpu.VMEM_SHARED((N + 8, HALF), jnp.bfloat16),
        pltpu.VMEM_SHARED((N + 8, DEG_W), jnp.bfloat16),
        pltpu.SemaphoreType.DMA,
        pltpu.SemaphoreType.DMA,
        pltpu.SemaphoreType.DMA,
        pltpu.SemaphoreType.DMA,
        pltpu.SemaphoreType.DMA,
        pltpu.SemaphoreType.DMA,
        pltpu.SemaphoreType.DMA,
        pltpu.SemaphoreType.DMA,
    ],
)(_sc_body)


ROW_BLK = 400


def _tc_body(acc0, deg0, acc1, deg1, w0, w1, b0, b1, out):
    h0 = acc0[:, :].astype(jnp.float32) / deg0[:, 0:1].astype(jnp.float32)
    h1 = acc1[:, :].astype(jnp.float32) / deg1[:, 0:1].astype(jnp.float32)
    y = (jnp.dot(h0, w0[:, :], preferred_element_type=jnp.float32)
         + jnp.dot(h1, w1[:, :], preferred_element_type=jnp.float32))
    out[:, :] = 0.5 * y + 0.5 * (b0[:, :] + b1[:, :])


def _tc_combine(acc0, deg0, acc1, deg1, w0, w1, b0, b1):
    grid = (N // ROW_BLK,)
    row_spec = pl.BlockSpec((ROW_BLK, D), lambda i: (i, 0))
    deg_spec = pl.BlockSpec((ROW_BLK, DEG_W), lambda i: (i, 0))
    full_spec = pl.BlockSpec((D, D), lambda i: (0, 0))
    bias_spec = pl.BlockSpec((1, D), lambda i: (0, 0))
    return pl.pallas_call(
        _tc_body,
        grid=grid,
        in_specs=[row_spec, deg_spec, row_spec, deg_spec,
                  full_spec, full_spec, bias_spec, bias_spec],
        out_specs=row_spec,
        out_shape=jax.ShapeDtypeStruct((N, D), jnp.float32),
    )(acc0, deg0, acc1, deg1, w0, w1, b0, b1)


def _pad_edges(edge_index):
    pad = PAD_CHUNKS * CHUNK - E
    src = jnp.concatenate(
        [edge_index[0], jnp.zeros((pad,), jnp.int32)]).reshape(PAD_CHUNKS, CHUNK)
    dst = jnp.concatenate(
        [edge_index[1], jnp.full((pad,), N, jnp.int32)]).reshape(PAD_CHUNKS, CHUNK)
    return src, dst


def kernel(x, edge_index_view0, edge_index_view1,
           W_view0, b_view0, W_view1, b_view1):
    src0, dst0 = _pad_edges(edge_index_view0)
    src1, dst1 = _pad_edges(edge_index_view1)
    x_bf = x.astype(jnp.bfloat16)
    acc0, deg0, acc1, deg1 = _sc_aggregate(x_bf, src0, dst0, src1, dst1)
    return _tc_combine(acc0, deg0, acc1, deg1,
                       W_view0, W_view1,
                       b_view0.reshape(1, D), b_view1.reshape(1, D))


# speedup vs baseline: 1.1093x; 1.0190x over previous
import functools

import jax
import jax.numpy as jnp
from jax import lax
from jax.experimental import pallas as pl
from jax.experimental.pallas import tpu as pltpu
from jax.experimental.pallas import tpu_sc as plsc

N = 10000
E = 320000
D = 128
HALF = 64

NS = 16
CHUNK = 128
PAD_CHUNKS = 2560
TILE_CHUNKS = PAD_CHUNKS // NS
DEG_W = 32

BLK = 80
NBLK = N // BLK
BLK_PER_TILE = -(-NBLK // NS)

NBUF = 8
NBODY = TILE_CHUNKS // NBUF


def _sc_body(x_hbm, src0, dst0, src1, dst1,
             acc0_hbm, deg0_hbm, acc1_hbm, deg1_hbm,
             src_v, dst_v, rows_v, ones_v, x_s, acc_s, deg_s,
             gs0, gs1, gs2, gs3, gs4, gs5, gs6, gs7,
             ss0, ss1, ss2, ss3, ss4, ss5, ss6, ss7):
    gsem = [gs0, gs1, gs2, gs3, gs4, gs5, gs6, gs7]
    ssem = [ss0, ss1, ss2, ss3, ss4, ss5, ss6, ss7]
    c = lax.axis_index("c")
    s = lax.axis_index("s")

    @pl.loop(0, CHUNK)
    def _fill(i):
        ones_v[i, :] = jnp.ones((DEG_W,), jnp.bfloat16)

    span = pl.multiple_of(s * TILE_CHUNKS, 8)

    @pl.when(c == 0)
    def _stage0():
        pltpu.sync_copy(src0.at[pl.ds(span, TILE_CHUNKS)], src_v)
        pltpu.sync_copy(dst0.at[pl.ds(span, TILE_CHUNKS)], dst_v)

    @pl.when(c == 1)
    def _stage1():
        pltpu.sync_copy(src1.at[pl.ds(span, TILE_CHUNKS)], src_v)
        pltpu.sync_copy(dst1.at[pl.ds(span, TILE_CHUNKS)], dst_v)

    for h in range(2):
        col = h * HALF

        @pl.loop(0, BLK_PER_TILE)
        def _init(k):
            bid = s * BLK_PER_TILE + k

            @pl.when(bid < NBLK)
            def _():
                off = pl.multiple_of(bid * BLK, 8)
                pltpu.sync_copy(
                    x_hbm.at[pl.ds(off, BLK), pl.ds(col, HALF)],
                    x_s.at[pl.ds(off, BLK)])
                pltpu.sync_copy(x_s.at[pl.ds(off, BLK)],
                                acc_s.at[pl.ds(off, BLK)])
                if h == 0:
                    pltpu.sync_copy(ones_v.at[pl.ds(0, BLK)],
                                    deg_s.at[pl.ds(off, BLK)])

        plsc.subcore_barrier()

        for b in range(NBUF):
            pltpu.async_copy(x_s.at[src_v.at[b]], rows_v.at[b], gsem[b])

        @pl.loop(0, NBODY)
        def _body(t):
            base = t * NBUF
            for b in range(NBUF):
                k = base + b
                pltpu.make_async_copy(
                    x_s.at[src_v.at[k]], rows_v.at[b], gsem[b]).wait()
                pltpu.async_copy(rows_v.at[b], acc_s.at[dst_v.at[k]],
                                 ssem[b], add=True)
                if h == 0:
                    pltpu.async_copy(ones_v, deg_s.at[dst_v.at[k]],
                                     ssem[b], add=True)
            for b in range(NBUF):
                k = base + b
                pltpu.make_async_copy(
                    rows_v.at[b], acc_s.at[dst_v.at[k]], ssem[b]).wait()
                if h == 0:
                    pltpu.make_async_copy(
                        ones_v, deg_s.at[dst_v.at[k]], ssem[b]).wait()

                @pl.when(t < NBODY - 1)
                def _():
                    kn = base + NBUF + b
                    pltpu.async_copy(x_s.at[src_v.at[kn]],
                                     rows_v.at[b], gsem[b])

        plsc.subcore_barrier()

        def copy_out(acc_hbm, deg_hbm):
            @pl.loop(0, BLK_PER_TILE)
            def _out(k):
                bid = s * BLK_PER_TILE + k

                @pl.when(bid < NBLK)
                def _():
                    off = pl.multiple_of(bid * BLK, 8)
                    pltpu.sync_copy(
                        acc_s.at[pl.ds(off, BLK)],
                        acc_hbm.at[pl.ds(off, BLK), pl.ds(col, HALF)])
                    if h == 0:
                        pltpu.sync_copy(deg_s.at[pl.ds(off, BLK)],
                                        deg_hbm.at[pl.ds(off, BLK)])

        @pl.when(c == 0)
        def _out0():
            copy_out(acc0_hbm, deg0_hbm)

        @pl.when(c == 1)
        def _out1():
            copy_out(acc1_hbm, deg1_hbm)


_sc_aggregate = functools.partial(
    pl.kernel,
    out_type=(
        jax.ShapeDtypeStruct((N, D), jnp.bfloat16),
        jax.ShapeDtypeStruct((N, DEG_W), jnp.bfloat16),
        jax.ShapeDtypeStruct((N, D), jnp.bfloat16),
        jax.ShapeDtypeStruct((N, DEG_W), jnp.bfloat16),
    ),
    mesh=plsc.VectorSubcoreMesh(core_axis_name="c", subcore_axis_name="s"),
    compiler_params=pltpu.CompilerParams(use_tc_tiling_on_sc=False),
    scratch_types=[
        pltpu.VMEM((TILE_CHUNKS, CHUNK), jnp.int32),
        pltpu.VMEM((TILE_CHUNKS, CHUNK), jnp.int32),
        pltpu.VMEM((NBUF, CHUNK, HALF), jnp.bfloat16),
        pltpu.VMEM((CHUNK, DEG_W), jnp.bfloat16),
        pltpu.VMEM_SHARED((N, HALF), jnp.bfloat16),
        pltpu.VMEM_SHARED((N + 8, HALF), jnp.bfloat16),
        pltpu.VMEM_SHARED((N + 8, DEG_W), jnp.bfloat16),
    ] + [pltpu.SemaphoreType.DMA] * 16,
)(_sc_body)


ROW_BLK = 400


def _tc_body(acc0, deg0, acc1, deg1, w0, w1, b0, b1, out):
    h0 = acc0[:, :].astype(jnp.float32) / deg0[:, 0:1].astype(jnp.float32)
    h1 = acc1[:, :].astype(jnp.float32) / deg1[:, 0:1].astype(jnp.float32)
    y = (jnp.dot(h0, w0[:, :], preferred_element_type=jnp.float32)
         + jnp.dot(h1, w1[:, :], preferred_element_type=jnp.float32))
    out[:, :] = 0.5 * y + 0.5 * (b0[:, :] + b1[:, :])


def _tc_combine(acc0, deg0, acc1, deg1, w0, w1, b0, b1):
    grid = (N // ROW_BLK,)
    row_spec = pl.BlockSpec((ROW_BLK, D), lambda i: (i, 0))
    deg_spec = pl.BlockSpec((ROW_BLK, DEG_W), lambda i: (i, 0))
    full_spec = pl.BlockSpec((D, D), lambda i: (0, 0))
    bias_spec = pl.BlockSpec((1, D), lambda i: (0, 0))
    return pl.pallas_call(
        _tc_body,
        grid=grid,
        in_specs=[row_spec, deg_spec, row_spec, deg_spec,
                  full_spec, full_spec, bias_spec, bias_spec],
        out_specs=row_spec,
        out_shape=jax.ShapeDtypeStruct((N, D), jnp.float32),
    )(acc0, deg0, acc1, deg1, w0, w1, b0, b1)


def _pad_edges(edge_index):
    pad = PAD_CHUNKS * CHUNK - E
    src = jnp.concatenate(
        [edge_index[0], jnp.zeros((pad,), jnp.int32)]).reshape(PAD_CHUNKS, CHUNK)
    dst = jnp.concatenate(
        [edge_index[1], jnp.full((pad,), N, jnp.int32)]).reshape(PAD_CHUNKS, CHUNK)
    return src, dst


def kernel(x, edge_index_view0, edge_index_view1,
           W_view0, b_view0, W_view1, b_view1):
    src0, dst0 = _pad_edges(edge_index_view0)
    src1, dst1 = _pad_edges(edge_index_view1)
    x_bf = x.astype(jnp.bfloat16)
    acc0, deg0, acc1, deg1 = _sc_aggregate(x_bf, src0, dst0, src1, dst1)
    return _tc_combine(acc0, deg0, acc1, deg1,
                       W_view0, W_view1,
                       b_view0.reshape(1, D), b_view1.reshape(1, D))
